# baseline (speedup 1.0000x reference)
import functools

import jax
import jax.numpy as jnp
from jax import lax
from jax.experimental import pallas as pl
from jax.experimental.pallas import tpu as pltpu
from jax.experimental.pallas import tpu_sc as plsc

B = 16384
SEQ = 50
VOCAB = 500
EMB = 32
H1 = 256
OUT = 10

SLOTS = 52
KPAD = SLOTS * EMB
NC = 2
NS = 16
NW = NC * NS

ROWS = B * SLOTS
ROWS_PER_W = ROWS // NW
CHUNK = 128
CHUNKS_PER_W = ROWS_PER_W // CHUNK
GROUP = 8
GROUPROWS = GROUP * CHUNK
GROUPS = CHUNKS_PER_W // GROUP


def _sc_gather(flat_idx, table):
    idx3 = flat_idx.reshape(NW, CHUNKS_PER_W, CHUNK)
    mesh = plsc.VectorSubcoreMesh(core_axis_name="c", subcore_axis_name="s")

    @functools.partial(
        pl.kernel,
        mesh=mesh,
        out_type=jax.ShapeDtypeStruct((ROWS, EMB), jnp.float32),
        scratch_types=[
            pltpu.VMEM((CHUNKS_PER_W, CHUNK), jnp.int32),
            pltpu.VMEM((GROUPROWS, EMB), jnp.float32),
            pltpu.VMEM((GROUPROWS, EMB), jnp.float32),
            pltpu.SemaphoreType.DMA,
            pltpu.SemaphoreType.DMA,
            pltpu.SemaphoreType.DMA,
        ],
        compiler_params=pltpu.CompilerParams(use_tc_tiling_on_sc=False),
    )
    def gather_kernel(idx_hbm, table_hbm, out_hbm, idx_v, rows0, rows1,
                      gsem, wsem0, wsem1):
        wid = lax.axis_index("s") * NC + lax.axis_index("c")
        base = wid * ROWS_PER_W
        bufs = (rows0, rows1)
        wsems = (wsem0, wsem1)
        pltpu.sync_copy(idx_hbm.at[wid], idx_v)

        def out_slice(g):
            return out_hbm.at[pl.ds(base + g * GROUPROWS, GROUPROWS)]

        def fill(g, buf):
            copies = []
            for b in range(GROUP):
                copies.append(
                    pltpu.async_copy(
                        table_hbm.at[idx_v.at[g * GROUP + b]],
                        buf.at[pl.ds(b * CHUNK, CHUNK)],
                        gsem,
                    )
                )
            for c in copies:
                c.wait()

        for p in range(2):
            fill(p, bufs[p])
            pltpu.async_copy(bufs[p], out_slice(p), wsems[p])

        def pair_body(i, carry):
            for p in range(2):
                g = 2 * i + p
                pltpu.make_async_copy(bufs[p], out_slice(g), wsems[p]).wait()
                fill(g, bufs[p])
                pltpu.async_copy(bufs[p], out_slice(g), wsems[p])
            return carry

        lax.fori_loop(1, GROUPS // 2, pair_body, 0)

        for p in range(2):
            g = GROUPS - 2 + p
            pltpu.make_async_copy(bufs[p], out_slice(g), wsems[p]).wait()

    return gather_kernel(idx3, table)


BT = 1024
BROWS = BT * KPAD // 128


def _mlp(emb128, W1p, b1, W2, b2):
    def body(e_ref, w1_ref, b1_ref, w2_ref, b2_ref, o_ref):
        e = e_ref[...].reshape(BT, KPAD)
        h = jnp.dot(e, w1_ref[...], preferred_element_type=jnp.float32)
        h = jnp.maximum(h + b1_ref[...], 0.0)
        logits = jnp.dot(h, w2_ref[...], preferred_element_type=jnp.float32)
        logits = logits + b2_ref[...]
        m = jnp.max(logits, axis=-1, keepdims=True)
        e2 = jnp.exp(logits - m)
        o_ref[...] = e2 / jnp.sum(e2, axis=-1, keepdims=True)

    return pl.pallas_call(
        body,
        grid=(B // BT,),
        in_specs=[
            pl.BlockSpec((BROWS, 128), lambda i: (i, 0)),
            pl.BlockSpec((KPAD, H1), lambda i: (0, 0)),
            pl.BlockSpec((1, H1), lambda i: (0, 0)),
            pl.BlockSpec((H1, OUT), lambda i: (0, 0)),
            pl.BlockSpec((1, OUT), lambda i: (0, 0)),
        ],
        out_specs=pl.BlockSpec((BT, OUT), lambda i: (i, 0)),
        out_shape=jax.ShapeDtypeStruct((B, OUT), jnp.float32),
    )(emb128, W1p, b1.reshape(1, H1), W2, b2.reshape(1, OUT))


def kernel(x, table, W1, b1, W2, b2):
    xp = jnp.concatenate(
        [x.astype(jnp.int32), jnp.zeros((B, SLOTS - SEQ), jnp.int32)], axis=1)
    emb = _sc_gather(xp.reshape(-1), table)
    emb128 = emb.reshape(B * KPAD // 128, 128)
    W1p = jnp.concatenate(
        [W1, jnp.zeros((KPAD - SEQ * EMB, H1), jnp.float32)], axis=0)
    return _mlp(emb128, W1p, b1, W2, b2)


# speedup vs baseline: 1.8418x; 1.8418x over previous
import functools

import jax
import jax.numpy as jnp
from jax import lax
from jax.experimental import pallas as pl
from jax.experimental.pallas import tpu as pltpu
from jax.experimental.pallas import tpu_sc as plsc

B = 16384
SEQ = 50
VOCAB = 500
EMB = 32
H1 = 256
OUT = 10

SLOTS = 52
KPAD = SLOTS * EMB
NC = 2
NS = 16
NW = NC * NS

ROWS = B * SLOTS
ROWS_PER_W = ROWS // NW
CHUNK = 128
CHUNKS_PER_W = ROWS_PER_W // CHUNK
GROUP = 8
GROUPROWS = GROUP * CHUNK
GROUPS = CHUNKS_PER_W // GROUP


def _sc_gather(flat_idx, table):
    idx3 = flat_idx.reshape(NW, CHUNKS_PER_W, CHUNK)
    mesh = plsc.VectorSubcoreMesh(core_axis_name="c", subcore_axis_name="s")

    @functools.partial(
        pl.kernel,
        mesh=mesh,
        out_type=jax.ShapeDtypeStruct((ROWS, EMB), jnp.float32),
        scratch_types=[
            pltpu.VMEM((CHUNKS_PER_W, CHUNK), jnp.int32),
            pltpu.VMEM((GROUPROWS, EMB), jnp.float32),
            pltpu.VMEM((GROUPROWS, EMB), jnp.float32),
            pltpu.SemaphoreType.DMA,
            pltpu.SemaphoreType.DMA,
            pltpu.SemaphoreType.DMA,
        ],
        compiler_params=pltpu.CompilerParams(use_tc_tiling_on_sc=False),
    )
    def gather_kernel(idx_hbm, table_hbm, out_hbm, idx_v, rows0, rows1,
                      gsem, wsem0, wsem1):
        wid = lax.axis_index("s") * NC + lax.axis_index("c")
        base = wid * ROWS_PER_W
        bufs = (rows0, rows1)
        wsems = (wsem0, wsem1)
        pltpu.sync_copy(idx_hbm.at[wid], idx_v)

        def out_slice(g):
            return out_hbm.at[pl.ds(base + g * GROUPROWS, GROUPROWS)]

        def fill(g, buf):
            copies = []
            for b in range(GROUP):
                copies.append(
                    pltpu.async_copy(
                        table_hbm.at[idx_v.at[g * GROUP + b]],
                        buf.at[pl.ds(b * CHUNK, CHUNK)],
                        gsem,
                    )
                )
            for c in copies:
                c.wait()

        for p in range(2):
            fill(p, bufs[p])
            pltpu.async_copy(bufs[p], out_slice(p), wsems[p])

        def pair_body(i, carry):
            for p in range(2):
                g = 2 * i + p
                pltpu.make_async_copy(bufs[p], out_slice(g), wsems[p]).wait()
                fill(g, bufs[p])
                pltpu.async_copy(bufs[p], out_slice(g), wsems[p])
            return carry

        lax.fori_loop(1, GROUPS // 2, pair_body, 0)

        for p in range(2):
            g = GROUPS - 2 + p
            pltpu.make_async_copy(bufs[p], out_slice(g), wsems[p]).wait()

    return gather_kernel(idx3, table)


BT = 1024
BROWS = BT * KPAD // 128


def _mlp(emb128, W1p, b1, W2, b2):
    def body(e_ref, w1_ref, b1_ref, w2_ref, b2_ref, o_ref):
        e = e_ref[...].reshape(BT, KPAD)
        h = jnp.dot(e, w1_ref[...], preferred_element_type=jnp.float32)
        h = jnp.maximum(h + b1_ref[...], 0.0)
        logits = jnp.dot(h, w2_ref[...], preferred_element_type=jnp.float32)
        logits = logits + b2_ref[...]
        m = jnp.max(logits, axis=-1, keepdims=True)
        e2 = jnp.exp(logits - m)
        o_ref[...] = e2 / jnp.sum(e2, axis=-1, keepdims=True)

    return pl.pallas_call(
        body,
        grid=(B // BT,),
        in_specs=[
            pl.BlockSpec((BROWS, 128), lambda i: (i, 0)),
            pl.BlockSpec((KPAD, H1), lambda i: (0, 0)),
            pl.BlockSpec((1, H1), lambda i: (0, 0)),
            pl.BlockSpec((H1, OUT), lambda i: (0, 0)),
            pl.BlockSpec((1, OUT), lambda i: (0, 0)),
        ],
        out_specs=pl.BlockSpec((BT, OUT), lambda i: (i, 0)),
        out_shape=jax.ShapeDtypeStruct((B, OUT), jnp.float32),
    )(emb128, W1p, b1.reshape(1, H1), W2, b2.reshape(1, OUT))


def kernel(x, table, W1, b1, W2, b2):
    xi = x.astype(jnp.int32)
    xp = jnp.concatenate([xi, xi[:, : SLOTS - SEQ]], axis=1)
    emb = _sc_gather(xp.reshape(-1), table)
    emb128 = emb.reshape(B * KPAD // 128, 128)
    W1p = jnp.concatenate(
        [W1, jnp.zeros((KPAD - SEQ * EMB, H1), jnp.float32)], axis=0)
    return _mlp(emb128, W1p, b1, W2, b2)
